# baseline (speedup 1.0000x reference)
import functools

import jax
import jax.numpy as jnp
from jax import lax
from jax.experimental import pallas as pl
from jax.experimental.pallas import tpu as pltpu
from jax.experimental.pallas import tpu_sc as plsc

BATCH = 4
SEQ = 2048
D = 1024
NC, NS, L = 2, 16, 16
NW = NC * NS
SB = SEQ // NW
CHUNK = 8
NCH = SB // CHUNK
NT = NCH * BATCH
NBUF = 8


def _emb_body(x_hbm, table_hbm, pe_hbm, out_hbm, idx_v, pe_v, rows_v,
              gsems, ssems, psems):
    wid = lax.axis_index("s") * NC + lax.axis_index("c")
    s0 = wid * SB

    def start_pe(c):
        src = pe_hbm.at[pl.ds(s0 + c * CHUNK, CHUNK)]
        return pltpu.async_copy(src, pe_v.at[c % 2], psems[c % 2])

    def start_gather(t):
        c, b = divmod(t, BATCH)
        idx_sl = idx_v.at[pl.ds(b * SB + c * CHUNK, CHUNK)]
        return pltpu.async_copy(table_hbm.at[idx_sl], rows_v.at[t % NBUF],
                                gsems[t % NBUF])

    def add_pe(t):
        c = t // BATCH
        k, kp = t % NBUF, c % 2
        vpr = D // L

        @plsc.parallel_loop(0, CHUNK * vpr, unroll=8)
        def _(i):
            r = i >> 6
            sl = pl.ds((i & (vpr - 1)) * L, L)
            plsc.addupdate(rows_v.at[k, r, sl], pe_v[kp, r, sl])

    def start_store(t):
        c, b = divmod(t, BATCH)
        dst = out_hbm.at[b, pl.ds(s0 + c * CHUNK, CHUNK)]
        return pltpu.async_copy(rows_v.at[t % NBUF], dst, ssems[t % NBUF])

    pe_d = {0: start_pe(0)}
    for b in range(BATCH):
        pltpu.sync_copy(x_hbm.at[b, pl.ds(s0, SB)],
                        idx_v.at[pl.ds(b * SB, SB)])
    gather_d = {t: start_gather(t) for t in range(NBUF - 1)}
    store_d = {}

    for t in range(NT):
        tg = t + NBUF - 1
        if tg < NT:
            if tg - NBUF >= 0:
                store_d.pop(tg - NBUF).wait()
            gather_d[tg] = start_gather(tg)
        c, b = divmod(t, BATCH)
        if b == 0:
            if c + 1 < NCH:
                pe_d[c + 1] = start_pe(c + 1)
            pe_d.pop(c).wait()
        gather_d.pop(t).wait()
        add_pe(t)
        store_d[t] = start_store(t)
    for d in store_d.values():
        d.wait()


@functools.partial(
    pl.kernel,
    mesh=plsc.VectorSubcoreMesh(core_axis_name="c", subcore_axis_name="s"),
    out_type=jax.ShapeDtypeStruct((BATCH, SEQ, D), jnp.float32),
    scratch_types=[
        pltpu.VMEM((BATCH * SB,), jnp.int32),
        pltpu.VMEM((2, CHUNK, D), jnp.float32),
        pltpu.VMEM((NBUF, CHUNK, D), jnp.float32),
        tuple(pltpu.SemaphoreType.DMA for _ in range(NBUF)),
        tuple(pltpu.SemaphoreType.DMA for _ in range(NBUF)),
        (pltpu.SemaphoreType.DMA, pltpu.SemaphoreType.DMA),
    ],
)
def _emb_call(x_hbm, table_hbm, pe_hbm, out_hbm, idx_v, pe_v, rows_v,
              gsems, ssems, psems):
    _emb_body(x_hbm, table_hbm, pe_hbm, out_hbm, idx_v, pe_v, rows_v,
              gsems, ssems, psems)


def kernel(x, table, pe):
    return _emb_call(x.astype(jnp.int32), table, pe)


# speedup vs baseline: 1.0226x; 1.0226x over previous
import functools

import jax
import jax.numpy as jnp
from jax import lax
from jax.experimental import pallas as pl
from jax.experimental.pallas import tpu as pltpu
from jax.experimental.pallas import tpu_sc as plsc

BATCH = 4
SEQ = 2048
D = 1024
NC, NS, L = 2, 16, 16
NW = NC * NS
SB = SEQ // NW
CHUNK = 16
NCH = SB // CHUNK
NT = NCH * BATCH
NBUF = 5


def _emb_body(x_hbm, table_hbm, pe_hbm, out_hbm, idx_v, pe_v, rows_v,
              gsems, ssems, psems):
    wid = lax.axis_index("s") * NC + lax.axis_index("c")
    s0 = wid * SB

    def start_pe(c):
        src = pe_hbm.at[pl.ds(s0 + c * CHUNK, CHUNK)]
        return pltpu.async_copy(src, pe_v.at[c % 2], psems[c % 2])

    def start_gather(t):
        c, b = divmod(t, BATCH)
        idx_sl = idx_v.at[pl.ds(b * SB + c * CHUNK, CHUNK)]
        return pltpu.async_copy(table_hbm.at[idx_sl], rows_v.at[t % NBUF],
                                gsems[t % NBUF])

    def add_pe(t):
        c = t // BATCH
        k, kp = t % NBUF, c % 2
        vpr = D // L

        @plsc.parallel_loop(0, CHUNK * vpr, unroll=8)
        def _(i):
            r = i >> 6
            sl = pl.ds((i & (vpr - 1)) * L, L)
            plsc.addupdate(rows_v.at[k, r, sl], pe_v[kp, r, sl])

    def start_store(t):
        c, b = divmod(t, BATCH)
        dst = out_hbm.at[b, pl.ds(s0 + c * CHUNK, CHUNK)]
        return pltpu.async_copy(rows_v.at[t % NBUF], dst, ssems[t % NBUF])

    pe_d = {0: start_pe(0)}
    for b in range(BATCH):
        pltpu.sync_copy(x_hbm.at[b, pl.ds(s0, SB)],
                        idx_v.at[pl.ds(b * SB, SB)])
    gather_d = {t: start_gather(t) for t in range(NBUF - 1)}
    store_d = {}

    for t in range(NT):
        tg = t + NBUF - 1
        if tg < NT:
            if tg - NBUF >= 0:
                store_d.pop(tg - NBUF).wait()
            gather_d[tg] = start_gather(tg)
        c, b = divmod(t, BATCH)
        if b == 0:
            if c + 1 < NCH:
                pe_d[c + 1] = start_pe(c + 1)
            pe_d.pop(c).wait()
        gather_d.pop(t).wait()
        add_pe(t)
        store_d[t] = start_store(t)
    for d in store_d.values():
        d.wait()


@functools.partial(
    pl.kernel,
    mesh=plsc.VectorSubcoreMesh(core_axis_name="c", subcore_axis_name="s"),
    out_type=jax.ShapeDtypeStruct((BATCH, SEQ, D), jnp.float32),
    scratch_types=[
        pltpu.VMEM((BATCH * SB,), jnp.int32),
        pltpu.VMEM((2, CHUNK, D), jnp.float32),
        pltpu.VMEM((NBUF, CHUNK, D), jnp.float32),
        tuple(pltpu.SemaphoreType.DMA for _ in range(NBUF)),
        tuple(pltpu.SemaphoreType.DMA for _ in range(NBUF)),
        (pltpu.SemaphoreType.DMA, pltpu.SemaphoreType.DMA),
    ],
)
def _emb_call(x_hbm, table_hbm, pe_hbm, out_hbm, idx_v, pe_v, rows_v,
              gsems, ssems, psems):
    _emb_body(x_hbm, table_hbm, pe_hbm, out_hbm, idx_v, pe_v, rows_v,
              gsems, ssems, psems)


def kernel(x, table, pe):
    return _emb_call(x.astype(jnp.int32), table, pe)


# speedup vs baseline: 1.0855x; 1.0615x over previous
import functools

import jax
import jax.numpy as jnp
from jax import lax
from jax.experimental import pallas as pl
from jax.experimental.pallas import tpu as pltpu
from jax.experimental.pallas import tpu_sc as plsc

BATCH = 4
SEQ = 2048
D = 1024
NC, NS, L = 2, 16, 16
NW = NC * NS
SB = SEQ // NW
CHUNK = 16
NCH = SB // CHUNK
NT = NCH * BATCH
NBUF = 5


def _emb_body(x_hbm, table_hbm, pe_hbm, out_hbm, idx_v, pe_v, rows_v,
              gsems, ssems, psems):
    wid = lax.axis_index("s") * NC + lax.axis_index("c")
    s0 = wid * SB

    def start_pe(c):
        src = pe_hbm.at[pl.ds(s0 + c * CHUNK, CHUNK)]
        return pltpu.async_copy(src, pe_v.at[c % 2], psems[c % 2])

    def start_gather(t):
        c, b = divmod(t, BATCH)
        idx_sl = idx_v.at[pl.ds(b * SB + c * CHUNK, CHUNK)]
        return pltpu.async_copy(table_hbm.at[idx_sl], rows_v.at[t % NBUF],
                                gsems[t % NBUF])

    def add_pe(t):
        c = t // BATCH
        k, kp = t % NBUF, c % 2
        vpr = D // L

        @plsc.parallel_loop(0, CHUNK * vpr, unroll=8)
        def _(i):
            r = i >> 6
            sl = pl.ds((i & (vpr - 1)) * L, L)
            plsc.addupdate(rows_v.at[k, r, sl], pe_v[kp, r, sl])

    def start_store(t):
        c, b = divmod(t, BATCH)
        dst = out_hbm.at[b, pl.ds(s0 + c * CHUNK, CHUNK)]
        return pltpu.async_copy(rows_v.at[t % NBUF], dst, ssems[t % NBUF])

    pe_d = {0: start_pe(0)}
    gather_d = {}
    for b in range(BATCH):
        pltpu.sync_copy(x_hbm.at[b, pl.ds(s0, SB)],
                        idx_v.at[pl.ds(b * SB, SB)])
        if b < NBUF - 1:
            gather_d[b] = start_gather(b)
    store_d = {}

    for t in range(NT):
        tg = t + NBUF - 1
        if tg < NT:
            if tg - NBUF >= 0:
                store_d.pop(tg - NBUF).wait()
            gather_d[tg] = start_gather(tg)
        c, b = divmod(t, BATCH)
        if b == 0:
            if c + 1 < NCH:
                pe_d[c + 1] = start_pe(c + 1)
            pe_d.pop(c).wait()
        gather_d.pop(t).wait()
        add_pe(t)
        store_d[t] = start_store(t)
    for d in store_d.values():
        d.wait()


@functools.partial(
    pl.kernel,
    mesh=plsc.VectorSubcoreMesh(core_axis_name="c", subcore_axis_name="s"),
    out_type=jax.ShapeDtypeStruct((BATCH, SEQ, D), jnp.float32),
    scratch_types=[
        pltpu.VMEM((BATCH * SB,), jnp.int32),
        pltpu.VMEM((2, CHUNK, D), jnp.float32),
        pltpu.VMEM((NBUF, CHUNK, D), jnp.float32),
        tuple(pltpu.SemaphoreType.DMA for _ in range(NBUF)),
        tuple(pltpu.SemaphoreType.DMA for _ in range(NBUF)),
        (pltpu.SemaphoreType.DMA, pltpu.SemaphoreType.DMA),
    ],
)
def _emb_call(x_hbm, table_hbm, pe_hbm, out_hbm, idx_v, pe_v, rows_v,
              gsems, ssems, psems):
    _emb_body(x_hbm, table_hbm, pe_hbm, out_hbm, idx_v, pe_v, rows_v,
              gsems, ssems, psems)


def kernel(x, table, pe):
    return _emb_call(x.astype(jnp.int32), table, pe)


# speedup vs baseline: 1.0961x; 1.0098x over previous
import functools

import jax
import jax.numpy as jnp
from jax import lax
from jax.experimental import pallas as pl
from jax.experimental.pallas import tpu as pltpu
from jax.experimental.pallas import tpu_sc as plsc

BATCH = 4
SEQ = 2048
D = 1024
NC, NS, L = 2, 16, 16
NW = NC * NS
SB = SEQ // NW
CHUNK = 16
NCH = SB // CHUNK
NT = NCH * BATCH
NBUF = 5


def _emb_body(x_hbm, table_hbm, pe_hbm, out_hbm, idx_v, pe_v, rows_v,
              gsems, ssems, psems):
    wid = lax.axis_index("s") * NC + lax.axis_index("c")
    s0 = wid * SB

    def start_pe(c):
        src = pe_hbm.at[pl.ds(s0 + c * CHUNK, CHUNK)]
        return pltpu.async_copy(src, pe_v.at[c % 2], psems[c % 2])

    def start_gather(t):
        c, b = divmod(t, BATCH)
        idx_sl = idx_v.at[pl.ds(b * SB + c * CHUNK, CHUNK)]
        return pltpu.async_copy(table_hbm.at[idx_sl], rows_v.at[t % NBUF],
                                gsems[t % NBUF], priority=1)

    def add_pe(t):
        c = t // BATCH
        k, kp = t % NBUF, c % 2
        vpr = D // L

        @plsc.parallel_loop(0, CHUNK * vpr, unroll=8)
        def _(i):
            r = i >> 6
            sl = pl.ds((i & (vpr - 1)) * L, L)
            plsc.addupdate(rows_v.at[k, r, sl], pe_v[kp, r, sl])

    def start_store(t):
        c, b = divmod(t, BATCH)
        dst = out_hbm.at[b, pl.ds(s0 + c * CHUNK, CHUNK)]
        return pltpu.async_copy(rows_v.at[t % NBUF], dst, ssems[t % NBUF])

    pe_d = {0: start_pe(0)}
    gather_d = {}
    for b in range(BATCH):
        pltpu.sync_copy(x_hbm.at[b, pl.ds(s0, SB)],
                        idx_v.at[pl.ds(b * SB, SB)])
        if b < NBUF - 1:
            gather_d[b] = start_gather(b)
    store_d = {}

    for t in range(NT):
        tg = t + NBUF - 1
        if tg < NT:
            if tg - NBUF >= 0:
                store_d.pop(tg - NBUF).wait()
            gather_d[tg] = start_gather(tg)
        c, b = divmod(t, BATCH)
        if b == 0:
            if c + 1 < NCH:
                pe_d[c + 1] = start_pe(c + 1)
            pe_d.pop(c).wait()
        gather_d.pop(t).wait()
        add_pe(t)
        store_d[t] = start_store(t)
    for d in store_d.values():
        d.wait()


@functools.partial(
    pl.kernel,
    mesh=plsc.VectorSubcoreMesh(core_axis_name="c", subcore_axis_name="s"),
    out_type=jax.ShapeDtypeStruct((BATCH, SEQ, D), jnp.float32),
    scratch_types=[
        pltpu.VMEM((BATCH * SB,), jnp.int32),
        pltpu.VMEM((2, CHUNK, D), jnp.float32),
        pltpu.VMEM((NBUF, CHUNK, D), jnp.float32),
        tuple(pltpu.SemaphoreType.DMA for _ in range(NBUF)),
        tuple(pltpu.SemaphoreType.DMA for _ in range(NBUF)),
        (pltpu.SemaphoreType.DMA, pltpu.SemaphoreType.DMA),
    ],
)
def _emb_call(x_hbm, table_hbm, pe_hbm, out_hbm, idx_v, pe_v, rows_v,
              gsems, ssems, psems):
    _emb_body(x_hbm, table_hbm, pe_hbm, out_hbm, idx_v, pe_v, rows_v,
              gsems, ssems, psems)


def kernel(x, table, pe):
    return _emb_call(x.astype(jnp.int32), table, pe)
